# baseline (speedup 1.0000x reference)
import jax
import jax.numpy as jnp
from jax.experimental import pallas as pl
from jax.experimental.pallas import tpu as pltpu

_BLK_S = 512
_EPS = 1e-5


def _ln_body(x_ref, pos_ref, g_ref, b_ref, o_ref):
    h = x_ref[...] + pos_ref[...]
    mean = jnp.mean(h, axis=-1, keepdims=True)
    d = h - mean
    var = jnp.mean(d * d, axis=-1, keepdims=True)
    o_ref[...] = d * jax.lax.rsqrt(var + _EPS) * g_ref[...] + b_ref[...]


def kernel(x, pos_table, gamma, beta):
    B, S, D = x.shape
    gamma2 = gamma.reshape(1, 1, D)
    beta2 = beta.reshape(1, 1, D)
    grid = (S // _BLK_S,)
    return pl.pallas_call(
        _ln_body,
        grid=grid,
        in_specs=[
            pl.BlockSpec((B, _BLK_S, D), lambda s: (0, s, 0)),
            pl.BlockSpec((1, _BLK_S, D), lambda s: (0, s, 0)),
            pl.BlockSpec((1, 1, D), lambda s: (0, 0, 0)),
            pl.BlockSpec((1, 1, D), lambda s: (0, 0, 0)),
        ],
        out_specs=pl.BlockSpec((B, _BLK_S, D), lambda s: (0, s, 0)),
        out_shape=jax.ShapeDtypeStruct((B, S, D), x.dtype),
        compiler_params=pltpu.CompilerParams(
            dimension_semantics=("parallel",),
        ),
    )(x, pos_table.reshape(1, S, D), gamma2, beta2)
